# Initial kernel scaffold; baseline (speedup 1.0000x reference)
import jax
import jax.numpy as jnp
from jax.experimental import pallas as pl


def kernel(x, table):
    raise NotImplementedError("write your pallas kernel here")


# speedup vs baseline: 2.6124x; 2.6124x over previous
import functools

import numpy as np
import jax
import jax.numpy as jnp
from jax import lax
from jax.experimental import pallas as pl
from jax.experimental.pallas import tpu as pltpu
from jax.experimental.pallas import tpu_sc as plsc

D_MODEL = 128
MAX_LEN = 200
HALF = 100
NUM_WORKERS = 32


def _positional_encoding(max_len, d_model):
    pos = jnp.arange(max_len, dtype=jnp.float32)[:, None]
    div = jnp.exp(
        jnp.arange(0, d_model, 2, dtype=jnp.float32)
        * (-(np.log(10000.0)) / d_model)
    )
    ang = pos * div
    pe = jnp.zeros((max_len, d_model), dtype=jnp.float32)
    pe = pe.at[:, 0::2].set(jnp.sin(ang))
    pe = pe.at[:, 1::2].set(jnp.cos(ang))
    return pe


@functools.lru_cache(maxsize=None)
def _make_kernel(n_flat, seq_len):
    rows_per_w = n_flat // NUM_WORKERS
    chunks = rows_per_w // seq_len
    halves_per_w = rows_per_w // HALF
    halves_per_chunk = seq_len // HALF

    mesh = plsc.VectorSubcoreMesh(core_axis_name="c", subcore_axis_name="s")

    @functools.partial(
        pl.kernel,
        mesh=mesh,
        out_type=jax.ShapeDtypeStruct((n_flat // HALF, HALF, D_MODEL), jnp.float32),
        scratch_types=[
            pltpu.VMEM((halves_per_chunk, HALF), jnp.int32),
            pltpu.VMEM((halves_per_chunk, HALF, D_MODEL), jnp.float32),
            pltpu.VMEM((halves_per_chunk, HALF, D_MODEL), jnp.float32),
            pltpu.SemaphoreType.DMA,
        ],
    )
    def k(idx_hbm, table_hbm, pe_hbm, out_hbm, idx_v, pe_v, rows_v, sem):
        wid = lax.axis_index("s") * 2 + lax.axis_index("c")
        pltpu.sync_copy(pe_hbm, pe_v)

        def chunk_body(c, carry):
            base = wid * halves_per_w + c * halves_per_chunk
            pltpu.sync_copy(idx_hbm.at[pl.ds(base, halves_per_chunk)], idx_v)
            cps = [
                pltpu.async_copy(table_hbm.at[idx_v.at[h]], rows_v.at[h], sem)
                for h in range(halves_per_chunk)
            ]
            for cp in cps:
                cp.wait()

            def row_body(r, carry2):
                for h in range(halves_per_chunk):
                    for j in range(D_MODEL // 16):
                        sl = pl.ds(j * 16, 16)
                        plsc.addupdate(rows_v.at[h, r, sl], pe_v[h, r, sl])
                return carry2

            lax.fori_loop(0, HALF, row_body, 0)
            pltpu.sync_copy(rows_v, out_hbm.at[pl.ds(base, halves_per_chunk)])
            return carry

        lax.fori_loop(0, chunks, chunk_body, 0)

    return k


def kernel(x, table):
    batch, seq_len = x.shape
    idx = x.reshape(-1, HALF).astype(jnp.int32)
    pe = _positional_encoding(MAX_LEN, D_MODEL)[:seq_len]
    pe = pe.reshape(seq_len // HALF, HALF, D_MODEL)
    k = _make_kernel(batch * seq_len, seq_len)
    out = k(idx, table, pe)
    return out.reshape(batch, seq_len, D_MODEL)


# speedup vs baseline: 3.8108x; 1.4587x over previous
import functools

import numpy as np
import jax
import jax.numpy as jnp
from jax import lax
from jax.experimental import pallas as pl
from jax.experimental.pallas import tpu as pltpu
from jax.experimental.pallas import tpu_sc as plsc

D_MODEL = 128
MAX_LEN = 200
CHUNK = 100
NBUF = 4
NUM_WORKERS = 32


def _positional_encoding(max_len, d_model):
    pos = jnp.arange(max_len, dtype=jnp.float32)[:, None]
    div = jnp.exp(
        jnp.arange(0, d_model, 2, dtype=jnp.float32)
        * (-(np.log(10000.0)) / d_model)
    )
    ang = pos * div
    pe = jnp.zeros((max_len, d_model), dtype=jnp.float32)
    pe = pe.at[:, 0::2].set(jnp.sin(ang))
    pe = pe.at[:, 1::2].set(jnp.cos(ang))
    return pe


@functools.lru_cache(maxsize=None)
def _make_kernel(n_flat, seq_len):
    rows_per_w = n_flat // NUM_WORKERS
    chunks = rows_per_w // CHUNK
    halves = seq_len // CHUNK

    mesh = plsc.VectorSubcoreMesh(core_axis_name="c", subcore_axis_name="s")

    @functools.partial(
        pl.kernel,
        mesh=mesh,
        out_type=jax.ShapeDtypeStruct((n_flat // CHUNK, CHUNK, D_MODEL), jnp.float32),
        scratch_types=[
            pltpu.VMEM((chunks, CHUNK), jnp.int32),
            pltpu.VMEM((halves, CHUNK, D_MODEL), jnp.float32),
            pltpu.VMEM((NBUF, CHUNK, D_MODEL), jnp.float32),
            pltpu.SemaphoreType.DMA((NBUF,)),
            pltpu.SemaphoreType.DMA((NBUF,)),
        ],
    )
    def k(idx_hbm, table_hbm, pe_hbm, out_hbm, idx_v, pe_v, rows_v, sem_g, sem_w):
        wid = lax.axis_index("s") * 2 + lax.axis_index("c")
        base = wid * chunks
        pltpu.sync_copy(pe_hbm, pe_v)
        pltpu.sync_copy(idx_hbm.at[pl.ds(base, chunks)], idx_v)

        for b in range(2):
            pltpu.async_copy(
                table_hbm.at[idx_v.at[b]], rows_v.at[b], sem_g.at[b]
            )

        def quad_body(p, carry):
            c0 = p * NBUF
            for b in range(NBUF):
                c = c0 + b
                n = c + 2
                nb = (b + 2) % NBUF

                @pl.when(n < chunks)
                def _():
                    @pl.when(n >= NBUF)
                    def _():
                        pltpu.make_async_copy(
                            rows_v.at[nb], out_hbm.at[0], sem_w.at[nb]
                        ).wait()

                    pltpu.async_copy(
                        table_hbm.at[idx_v.at[n]], rows_v.at[nb], sem_g.at[nb]
                    )

                pltpu.make_async_copy(
                    out_hbm.at[0], rows_v.at[b], sem_g.at[b]
                ).wait()

                def row_body(r, carry2):
                    for j in range(D_MODEL // 16):
                        sl = pl.ds(j * 16, 16)
                        plsc.addupdate(rows_v.at[b, r, sl], pe_v[b % halves, r, sl])
                    return carry2

                lax.fori_loop(0, CHUNK, row_body, 0)
                pltpu.async_copy(rows_v.at[b], out_hbm.at[base + c], sem_w.at[b])
            return carry

        lax.fori_loop(0, chunks // NBUF, quad_body, 0)

        for b in range(NBUF):
            pltpu.make_async_copy(
                rows_v.at[b], out_hbm.at[0], sem_w.at[b]
            ).wait()

    return k


def kernel(x, table):
    batch, seq_len = x.shape
    idx = x.reshape(-1, CHUNK).astype(jnp.int32)
    pe = _positional_encoding(MAX_LEN, D_MODEL)[:seq_len]
    pe = pe.reshape(seq_len // CHUNK, CHUNK, D_MODEL)
    k = _make_kernel(batch * seq_len, seq_len)
    out = k(idx, table, pe)
    return out.reshape(batch, seq_len, D_MODEL)


# speedup vs baseline: 6.3746x; 1.6728x over previous
import functools

import numpy as np
import jax
import jax.numpy as jnp
from jax import lax
from jax.experimental import pallas as pl
from jax.experimental.pallas import tpu as pltpu
from jax.experimental.pallas import tpu_sc as plsc

D_MODEL = 128
MAX_LEN = 200
IDXW = 100
NBUF = 2
NUM_WORKERS = 32


def _positional_encoding(max_len, d_model):
    pos = jnp.arange(max_len, dtype=jnp.float32)[:, None]
    div = jnp.exp(
        jnp.arange(0, d_model, 2, dtype=jnp.float32)
        * (-(np.log(10000.0)) / d_model)
    )
    ang = pos * div
    pe = jnp.zeros((max_len, d_model), dtype=jnp.float32)
    pe = pe.at[:, 0::2].set(jnp.sin(ang))
    pe = pe.at[:, 1::2].set(jnp.cos(ang))
    return pe


@functools.lru_cache(maxsize=None)
def _make_kernel(batch, seq_len):
    seqs_per_w = batch // NUM_WORKERS
    halves = seq_len // IDXW
    idx_rows = seqs_per_w * halves

    mesh = plsc.VectorSubcoreMesh(core_axis_name="c", subcore_axis_name="s")

    @functools.partial(
        pl.kernel,
        mesh=mesh,
        out_type=jax.ShapeDtypeStruct((batch, seq_len, D_MODEL), jnp.float32),
        scratch_types=[
            pltpu.VMEM((idx_rows, IDXW), jnp.int32),
            pltpu.VMEM((seq_len, D_MODEL), jnp.float32),
            pltpu.VMEM((NBUF, seq_len, D_MODEL), jnp.float32),
            pltpu.SemaphoreType.DMA((NBUF,)),
            pltpu.SemaphoreType.DMA((NBUF,)),
        ],
    )
    def k(idx_hbm, table_hbm, pe_hbm, out_hbm, idx_v, pe_v, rows_v, sem_g, sem_w):
        wid = lax.axis_index("s") * 2 + lax.axis_index("c")
        bbase = wid * seqs_per_w
        pltpu.sync_copy(pe_hbm, pe_v)
        pltpu.sync_copy(idx_hbm.at[pl.ds(wid * idx_rows, idx_rows)], idx_v)

        def gather_seq(c, b):
            for h in range(halves):
                pltpu.async_copy(
                    table_hbm.at[idx_v.at[halves * c + h]],
                    rows_v.at[b, pl.ds(h * IDXW, IDXW)],
                    sem_g.at[b],
                )

        gather_seq(0, 0)

        def pair_body(p, carry):
            for b in range(NBUF):
                c = NBUF * p + b
                n = c + 1
                nb = (b + 1) % NBUF

                @pl.when(n < seqs_per_w)
                def _():
                    @pl.when(n >= NBUF)
                    def _():
                        pltpu.make_async_copy(
                            rows_v.at[nb], out_hbm.at[0], sem_w.at[nb]
                        ).wait()

                    gather_seq(n, nb)

                pltpu.make_async_copy(
                    out_hbm.at[0], rows_v.at[b], sem_g.at[b]
                ).wait()

                def row_body(r, carry2):
                    for u in range(2):
                        for j in range(D_MODEL // 16):
                            sl = pl.ds(j * 16, 16)
                            plsc.addupdate(
                                rows_v.at[b, 2 * r + u, sl], pe_v[2 * r + u, sl]
                            )
                    return carry2

                lax.fori_loop(0, seq_len // 2, row_body, 0)
                pltpu.async_copy(rows_v.at[b], out_hbm.at[bbase + c], sem_w.at[b])
            return carry

        lax.fori_loop(0, seqs_per_w // NBUF, pair_body, 0)

        for b in range(NBUF):
            pltpu.make_async_copy(
                rows_v.at[b], out_hbm.at[0], sem_w.at[b]
            ).wait()

    return k


def kernel(x, table):
    batch, seq_len = x.shape
    idx = x.reshape(-1, IDXW).astype(jnp.int32)
    pe = _positional_encoding(MAX_LEN, D_MODEL)[:seq_len]
    k = _make_kernel(batch, seq_len)
    return k(idx, table, pe)


# speedup vs baseline: 6.3847x; 1.0016x over previous
import functools

import numpy as np
import jax
import jax.numpy as jnp
from jax import lax
from jax.experimental import pallas as pl
from jax.experimental.pallas import tpu as pltpu
from jax.experimental.pallas import tpu_sc as plsc

D_MODEL = 128
MAX_LEN = 200
IDXW = 100
NBUF = 3
NUM_WORKERS = 32


def _positional_encoding(max_len, d_model):
    pos = jnp.arange(max_len, dtype=jnp.float32)[:, None]
    div = jnp.exp(
        jnp.arange(0, d_model, 2, dtype=jnp.float32)
        * (-(np.log(10000.0)) / d_model)
    )
    ang = pos * div
    pe = jnp.zeros((max_len, d_model), dtype=jnp.float32)
    pe = pe.at[:, 0::2].set(jnp.sin(ang))
    pe = pe.at[:, 1::2].set(jnp.cos(ang))
    return pe


@functools.lru_cache(maxsize=None)
def _make_kernel(batch, seq_len):
    seqs_per_w = batch // NUM_WORKERS
    halves = seq_len // IDXW
    idx_rows = seqs_per_w * halves

    mesh = plsc.VectorSubcoreMesh(core_axis_name="c", subcore_axis_name="s")

    @functools.partial(
        pl.kernel,
        mesh=mesh,
        out_type=jax.ShapeDtypeStruct((batch, seq_len, D_MODEL), jnp.float32),
        scratch_types=[
            pltpu.VMEM((idx_rows, IDXW), jnp.int32),
            pltpu.VMEM((seq_len, D_MODEL), jnp.float32),
            pltpu.VMEM((NBUF, seq_len, D_MODEL), jnp.float32),
            pltpu.SemaphoreType.DMA((NBUF,)),
            pltpu.SemaphoreType.DMA((NBUF,)),
        ],
    )
    def k(idx_hbm, table_hbm, pe_hbm, out_hbm, idx_v, pe_v, rows_v, sem_g, sem_w):
        wid = lax.axis_index("s") * 2 + lax.axis_index("c")
        bbase = wid * seqs_per_w
        pltpu.sync_copy(pe_hbm, pe_v)
        pltpu.sync_copy(idx_hbm.at[pl.ds(wid * idx_rows, idx_rows)], idx_v)

        def gather_seq(c, b):
            for h in range(halves):
                pltpu.async_copy(
                    table_hbm.at[idx_v.at[halves * c + h]],
                    rows_v.at[b, pl.ds(h * IDXW, IDXW)],
                    sem_g.at[b],
                )

        gather_seq(0, 0)
        gather_seq(1, 1)

        def tri_body(p, carry):
            for b in range(NBUF):
                c = NBUF * p + b

                @pl.when(c < seqs_per_w)
                def _():
                    n = c + 2
                    nb = (b + 2) % NBUF

                    @pl.when(n < seqs_per_w)
                    def _():
                        @pl.when(n >= NBUF)
                        def _():
                            pltpu.make_async_copy(
                                rows_v.at[nb], out_hbm.at[0], sem_w.at[nb]
                            ).wait()

                        gather_seq(n, nb)

                    pltpu.make_async_copy(
                        out_hbm.at[0], rows_v.at[b], sem_g.at[b]
                    ).wait()

                    def row_body(r, carry2):
                        for u in range(2):
                            for j in range(D_MODEL // 16):
                                sl = pl.ds(j * 16, 16)
                                plsc.addupdate(
                                    rows_v.at[b, 2 * r + u, sl], pe_v[2 * r + u, sl]
                                )
                        return carry2

                    lax.fori_loop(0, seq_len // 2, row_body, 0)
                    pltpu.async_copy(
                        rows_v.at[b], out_hbm.at[bbase + c], sem_w.at[b]
                    )
            return carry

        lax.fori_loop(0, (seqs_per_w + NBUF - 1) // NBUF, tri_body, 0)

        for b in range(NBUF):
            pltpu.make_async_copy(
                rows_v.at[b], out_hbm.at[0], sem_w.at[b]
            ).wait()

    return k


def kernel(x, table):
    batch, seq_len = x.shape
    idx = x.reshape(-1, IDXW).astype(jnp.int32)
    pe = _positional_encoding(MAX_LEN, D_MODEL)[:seq_len]
    k = _make_kernel(batch, seq_len)
    return k(idx, table, pe)
